# baseline (speedup 1.0000x reference)
import functools

import jax
import jax.numpy as jnp
from jax import lax
from jax.experimental import pallas as pl
from jax.experimental.pallas import tpu as pltpu
from jax.experimental.pallas import tpu_sc as plsc

BATCH = 16384
HIST = 50
EMBED = 32
VOCAB = 1000000

NC = 2
NS = 16
NW = NC * NS

NBT = BATCH // 128
NFT = EMBED // 8
BT_PER_W = NBT // NW
NCELL = HIST * BT_PER_W

NT_FULL = VOCAB // 128
T_PER_W = NT_FULL // NW
T_EXTRA = NT_FULL - T_PER_W * NW
V_TAIL = VOCAB - NT_FULL * 128

G = 2
NIT = T_PER_W // G

_mesh = plsc.VectorSubcoreMesh(
    core_axis_name="c", subcore_axis_name="s", num_cores=NC, num_subcores=NS
)


def _iota16():
    return lax.iota(jnp.int32, 16)


def _full16(v):
    return jnp.full((16,), v, jnp.int32)


def _transpose_block(src, dst, n_rows, col_base, iota):
    for i in range(n_rows):
        for k in range(8):
            v = plsc.load_gather(
                src, [iota + 16 * (k % 2), _full16(col_base + 4 * i + k // 2)]
            )
            dst[i, pl.ds(16 * k, 16)] = v


@functools.partial(
    pl.kernel,
    mesh=_mesh,
    compiler_params=pltpu.CompilerParams(
        use_tc_tiling_on_sc=True, needs_layout_passes=False
    ),
    out_type=jax.ShapeDtypeStruct((VOCAB // 4, 128), jnp.float32),
    scratch_types=[
        pltpu.VMEM((2, EMBED, G * 128), jnp.float32),
        pltpu.VMEM((2, G * 32, 128), jnp.float32),
        pltpu.VMEM((EMBED, 128), jnp.float32),
        pltpu.VMEM((32, 128), jnp.float32),
        pltpu.VMEM((EMBED, V_TAIL), jnp.float32),
        pltpu.VMEM((V_TAIL * EMBED // 128, 128), jnp.float32),
        pltpu.SemaphoreType.DMA,
        pltpu.SemaphoreType.DMA,
    ],
)
def _transpose_kernel(
    tab_hbm, scr_hbm, in_v, out_v, ex_in, ex_out, tail_in, tail_out, isem, osem
):
    wid = lax.axis_index("s") * NC + lax.axis_index("c")
    wbase = wid * T_PER_W
    iota = _iota16()

    def in_src(it):
        return tab_hbm.at[:, pl.ds((wbase + it * G) * 128, G * 128)]

    def out_dst(it):
        return scr_hbm.at[pl.ds((wbase + it * G) * 32, G * 32), :]

    pltpu.async_copy(in_src(0), in_v.at[0], isem)

    @pl.loop(0, NIT, step=2)
    def _steps(o):
        for p in range(2):
            it = o + p
            pltpu.make_async_copy(in_src(it), in_v.at[p], isem).wait()

            @pl.when(it + 1 < NIT)
            def _prefetch():
                pltpu.async_copy(in_src(it + 1), in_v.at[1 - p], isem)

            @pl.when(it >= 2)
            def _drain():
                pltpu.make_async_copy(out_v.at[p], out_dst(it - 2), osem).wait()

            src = in_v.at[p]

            def _row(r, _p=p, _src=src):
                base = 128 * (r // 32) + 4 * (r % 32)
                for k in range(8):
                    v = plsc.load_gather(
                        _src,
                        [iota + 16 * (k % 2), jnp.full((16,), base + k // 2, jnp.int32)],
                    )
                    out_v[_p, r, pl.ds(16 * k, 16)] = v

            plsc.parallel_loop(0, G * 32, unroll=8)(_row)
            pltpu.async_copy(out_v.at[p], out_dst(it), osem)

    pltpu.make_async_copy(out_v.at[0], out_dst(NIT - 2), osem).wait()
    pltpu.make_async_copy(out_v.at[1], out_dst(NIT - 1), osem).wait()

    @pl.when(wid < T_EXTRA)
    def _extra():
        t = NW * T_PER_W + wid
        pltpu.sync_copy(tab_hbm.at[:, pl.ds(t * 128, 128)], ex_in)
        _transpose_block(ex_in, ex_out, 32, 0, iota)
        pltpu.sync_copy(ex_out, scr_hbm.at[pl.ds(t * 32, 32), :])

    @pl.when(wid == T_EXTRA)
    def _tail():
        pltpu.sync_copy(tab_hbm.at[:, pl.ds(NT_FULL * 128, V_TAIL)], tail_in)
        _transpose_block(tail_in, tail_out, V_TAIL * EMBED // 128, 0, iota)
        pltpu.sync_copy(
            tail_out, scr_hbm.at[pl.ds(NT_FULL * 32, V_TAIL * EMBED // 128), :]
        )


@functools.partial(
    pl.kernel,
    mesh=_mesh,
    compiler_params=pltpu.CompilerParams(
        use_tc_tiling_on_sc=False, needs_layout_passes=False
    ),
    out_type=jax.ShapeDtypeStruct((HIST, NFT, NBT, 8, 128), jnp.float32),
    scratch_types=[
        pltpu.VMEM((HIST, BT_PER_W, 128), jnp.int32),
        pltpu.VMEM((2, 128, EMBED), jnp.float32),
        pltpu.VMEM((2, EMBED, 128), jnp.float32),
        pltpu.SemaphoreType.DMA,
        pltpu.SemaphoreType.DMA,
    ],
)
def _gather_kernel(idx_hbm, tab_hbm, out_hbm, idx_v, rows_v, trans_v, gsem, osem):
    wid = lax.axis_index("s") * NC + lax.axis_index("c")
    iota = _iota16()

    pltpu.sync_copy(idx_hbm.at[:, pl.ds(wid * BT_PER_W, BT_PER_W), :], idx_v)

    def gsrc(q):
        return tab_hbm.at[idx_v.at[q // BT_PER_W, q % BT_PER_W]]

    pltpu.async_copy(gsrc(0), rows_v.at[0], gsem)

    @pl.loop(0, NCELL, step=2)
    def _cells(o):
        for p in range(2):
            q = o + p
            h = q // BT_PER_W
            bt = wid * BT_PER_W + q % BT_PER_W
            pltpu.make_async_copy(gsrc(q), rows_v.at[p], gsem).wait()

            @pl.when(q + 1 < NCELL)
            def _prefetch():
                pltpu.async_copy(gsrc(q + 1), rows_v.at[1 - p], gsem)

            @pl.when(q >= 2)
            def _drain():
                hd = (q - 2) // BT_PER_W
                btd = wid * BT_PER_W + (q - 2) % BT_PER_W
                for ft in range(NFT):
                    pltpu.make_async_copy(
                        trans_v.at[p].at[pl.ds(8 * ft, 8)],
                        out_hbm.at[hd, ft, btd],
                        osem,
                    ).wait()

            rsrc = rows_v.at[p]

            def _feat(f, _p=p, _src=rsrc):
                for c in range(8):
                    v = plsc.load_gather(
                        _src, [iota + 16 * c, jnp.full((16,), f, jnp.int32)]
                    )
                    trans_v[_p, f, pl.ds(16 * c, 16)] = v

            plsc.parallel_loop(0, EMBED, unroll=8)(_feat)
            for ft in range(NFT):
                pltpu.async_copy(
                    trans_v.at[p].at[pl.ds(8 * ft, 8)], out_hbm.at[h, ft, bt], osem
                )

    for p in range(2):
        q = NCELL - 2 + p
        for ft in range(NFT):
            pltpu.make_async_copy(
                trans_v.at[p].at[pl.ds(8 * ft, 8)],
                out_hbm.at[q // BT_PER_W, ft, wid * BT_PER_W + q % BT_PER_W],
                osem,
            ).wait()


def kernel(indices, table):
    table_t = jnp.swapaxes(table, 0, 1)
    table_lin = _transpose_kernel(table_t).reshape(VOCAB, EMBED)
    idx_lin = (
        jnp.swapaxes(indices, 0, 1).astype(jnp.int32).reshape(HIST, NBT, 128)
    )
    out5 = _gather_kernel(idx_lin, table_lin)
    return out5.transpose(2, 4, 0, 1, 3).reshape(BATCH, HIST, EMBED)


# speedup vs baseline: 1.0360x; 1.0360x over previous
import functools

import jax
import jax.numpy as jnp
from jax import lax
from jax.experimental import pallas as pl
from jax.experimental.pallas import tpu as pltpu
from jax.experimental.pallas import tpu_sc as plsc

BATCH = 16384
HIST = 50
EMBED = 32
VOCAB = 1000000

NC = 2
NS = 16
NW = NC * NS

NBT = BATCH // 128
NFT = EMBED // 8
BT_PER_W = NBT // NW
NCELL = HIST * BT_PER_W

NT_FULL = VOCAB // 128
T_PER_W = NT_FULL // NW
T_EXTRA = NT_FULL - T_PER_W * NW
V_TAIL = VOCAB - NT_FULL * 128

G = 1
NIT = T_PER_W // G

_mesh = plsc.VectorSubcoreMesh(
    core_axis_name="c", subcore_axis_name="s", num_cores=NC, num_subcores=NS
)


def _iota16():
    return lax.iota(jnp.int32, 16)


def _full16(v):
    return jnp.full((16,), v, jnp.int32)


def _transpose_block(src, dst, n_rows, col_base, iota):
    for i in range(n_rows):
        for k in range(8):
            v = plsc.load_gather(
                src, [iota + 16 * (k % 2), _full16(col_base + 4 * i + k // 2)]
            )
            dst[i, pl.ds(16 * k, 16)] = v


@functools.partial(
    pl.kernel,
    mesh=_mesh,
    compiler_params=pltpu.CompilerParams(
        use_tc_tiling_on_sc=True, needs_layout_passes=False
    ),
    out_type=jax.ShapeDtypeStruct((VOCAB // 4, 128), jnp.float32),
    scratch_types=[
        pltpu.VMEM((4, EMBED, G * 128), jnp.float32),
        pltpu.VMEM((4, G * 32, 128), jnp.float32),
        pltpu.VMEM((EMBED, 128), jnp.float32),
        pltpu.VMEM((32, 128), jnp.float32),
        pltpu.VMEM((EMBED, V_TAIL), jnp.float32),
        pltpu.VMEM((V_TAIL * EMBED // 128, 128), jnp.float32),
        pltpu.SemaphoreType.DMA,
        pltpu.SemaphoreType.DMA,
    ],
)
def _transpose_kernel(
    tab_hbm, scr_hbm, in_v, out_v, ex_in, ex_out, tail_in, tail_out, isem, osem
):
    wid = lax.axis_index("s") * NC + lax.axis_index("c")
    wbase = wid * T_PER_W
    iota = _iota16()

    def in_src(it):
        return tab_hbm.at[:, pl.ds((wbase + it * G) * 128, G * 128)]

    def out_dst(it):
        return scr_hbm.at[pl.ds((wbase + it * G) * 32, G * 32), :]

    for w in range(3):
        pltpu.async_copy(in_src(w), in_v.at[w], isem)

    @pl.loop(0, NIT, step=4)
    def _steps(o):
        for p in range(4):
            it = o + p
            pltpu.make_async_copy(in_src(it), in_v.at[p], isem).wait()

            @pl.when(it + 3 < NIT)
            def _prefetch():
                pltpu.async_copy(in_src(it + 3), in_v.at[(p + 3) % 4], isem)

            @pl.when(it >= 4)
            def _drain():
                pltpu.make_async_copy(out_v.at[p], out_dst(it - 4), osem).wait()

            src = in_v.at[p]

            def _row(r, _p=p, _src=src):
                base = 128 * (r // 32) + 4 * (r % 32)
                for k in range(8):
                    v = plsc.load_gather(
                        _src,
                        [iota + 16 * (k % 2), jnp.full((16,), base + k // 2, jnp.int32)],
                    )
                    out_v[_p, r, pl.ds(16 * k, 16)] = v

            plsc.parallel_loop(0, G * 32, unroll=4)(_row)
            pltpu.async_copy(out_v.at[p], out_dst(it), osem)

    for p in range(4):
        pltpu.make_async_copy(out_v.at[p], out_dst(NIT - 4 + p), osem).wait()

    @pl.when(wid < T_EXTRA)
    def _extra():
        t = NW * T_PER_W + wid
        pltpu.sync_copy(tab_hbm.at[:, pl.ds(t * 128, 128)], ex_in)
        _transpose_block(ex_in, ex_out, 32, 0, iota)
        pltpu.sync_copy(ex_out, scr_hbm.at[pl.ds(t * 32, 32), :])

    @pl.when(wid == T_EXTRA)
    def _tail():
        pltpu.sync_copy(tab_hbm.at[:, pl.ds(NT_FULL * 128, V_TAIL)], tail_in)
        _transpose_block(tail_in, tail_out, V_TAIL * EMBED // 128, 0, iota)
        pltpu.sync_copy(
            tail_out, scr_hbm.at[pl.ds(NT_FULL * 32, V_TAIL * EMBED // 128), :]
        )


@functools.partial(
    pl.kernel,
    mesh=_mesh,
    compiler_params=pltpu.CompilerParams(
        use_tc_tiling_on_sc=False, needs_layout_passes=False
    ),
    out_type=jax.ShapeDtypeStruct((HIST, NFT, NBT, 8, 128), jnp.float32),
    scratch_types=[
        pltpu.VMEM((HIST, BT_PER_W, 128), jnp.int32),
        pltpu.VMEM((4, 128, EMBED), jnp.float32),
        pltpu.VMEM((4, EMBED, 128), jnp.float32),
        pltpu.SemaphoreType.DMA,
        pltpu.SemaphoreType.DMA,
    ],
)
def _gather_kernel(idx_hbm, tab_hbm, out_hbm, idx_v, rows_v, trans_v, gsem, osem):
    wid = lax.axis_index("s") * NC + lax.axis_index("c")
    iota = _iota16()

    pltpu.sync_copy(idx_hbm.at[:, pl.ds(wid * BT_PER_W, BT_PER_W), :], idx_v)

    def gsrc(q):
        return tab_hbm.at[idx_v.at[q // BT_PER_W, q % BT_PER_W]]

    for w in range(3):
        pltpu.async_copy(gsrc(w), rows_v.at[w], gsem)

    @pl.loop(0, NCELL, step=4)
    def _cells(o):
        for p in range(4):
            q = o + p
            h = q // BT_PER_W
            bt = wid * BT_PER_W + q % BT_PER_W
            pltpu.make_async_copy(gsrc(q), rows_v.at[p], gsem).wait()

            @pl.when(q + 3 < NCELL)
            def _prefetch():
                pltpu.async_copy(gsrc(q + 3), rows_v.at[(p + 3) % 4], gsem)

            @pl.when(q >= 4)
            def _drain():
                hd = (q - 4) // BT_PER_W
                btd = wid * BT_PER_W + (q - 4) % BT_PER_W
                for ft in range(NFT):
                    pltpu.make_async_copy(
                        trans_v.at[p].at[pl.ds(8 * ft, 8)],
                        out_hbm.at[hd, ft, btd],
                        osem,
                    ).wait()

            rsrc = rows_v.at[p]

            def _feat(f, _p=p, _src=rsrc):
                for c in range(8):
                    v = plsc.load_gather(
                        _src, [iota + 16 * c, jnp.full((16,), f, jnp.int32)]
                    )
                    trans_v[_p, f, pl.ds(16 * c, 16)] = v

            plsc.parallel_loop(0, EMBED, unroll=4)(_feat)
            for ft in range(NFT):
                pltpu.async_copy(
                    trans_v.at[p].at[pl.ds(8 * ft, 8)], out_hbm.at[h, ft, bt], osem
                )

    for p in range(4):
        q = NCELL - 4 + p
        for ft in range(NFT):
            pltpu.make_async_copy(
                trans_v.at[p].at[pl.ds(8 * ft, 8)],
                out_hbm.at[q // BT_PER_W, ft, wid * BT_PER_W + q % BT_PER_W],
                osem,
            ).wait()


def kernel(indices, table):
    table_t = jnp.swapaxes(table, 0, 1)
    table_lin = _transpose_kernel(table_t).reshape(VOCAB, EMBED)
    idx_lin = (
        jnp.swapaxes(indices, 0, 1).astype(jnp.int32).reshape(HIST, NBT, 128)
    )
    out5 = _gather_kernel(idx_lin, table_lin)
    return out5.transpose(2, 4, 0, 1, 3).reshape(BATCH, HIST, EMBED)


# speedup vs baseline: 2.6232x; 2.5320x over previous
import functools

import jax
import jax.numpy as jnp
from jax import lax
from jax.experimental import pallas as pl
from jax.experimental.pallas import tpu as pltpu
from jax.experimental.pallas import tpu_sc as plsc

BATCH = 16384
HIST = 50
EMBED = 32
VOCAB = 1000000

NC = 2
NS = 16
NW = NC * NS

NBT = BATCH // 128
NFT = EMBED // 8
BT_PER_W = NBT // NW
NCELL = HIST * BT_PER_W

NT_FULL = VOCAB // 128
T_PER_W = NT_FULL // NW
T_EXTRA = NT_FULL - T_PER_W * NW
V_TAIL = VOCAB - NT_FULL * 128

G = 1
NIT = T_PER_W // G

_mesh = plsc.VectorSubcoreMesh(
    core_axis_name="c", subcore_axis_name="s", num_cores=NC, num_subcores=NS
)


def _iota16():
    return lax.iota(jnp.int32, 16)


def _full16(v):
    return jnp.full((16,), v, jnp.int32)


def _transpose_block(src, dst, n_rows, col_base, iota):
    for i in range(n_rows):
        for k in range(8):
            v = plsc.load_gather(
                src, [iota + 16 * (k % 2), _full16(col_base + 4 * i + k // 2)]
            )
            dst[i, pl.ds(16 * k, 16)] = v


@functools.partial(
    pl.kernel,
    mesh=_mesh,
    compiler_params=pltpu.CompilerParams(
        use_tc_tiling_on_sc=True, needs_layout_passes=False
    ),
    out_type=jax.ShapeDtypeStruct((VOCAB // 4, 128), jnp.float32),
    scratch_types=[
        pltpu.VMEM((4, EMBED, G * 128), jnp.float32),
        pltpu.VMEM((4, G * 32, 128), jnp.float32),
        pltpu.VMEM((EMBED, 128), jnp.float32),
        pltpu.VMEM((32, 128), jnp.float32),
        pltpu.VMEM((EMBED, V_TAIL), jnp.float32),
        pltpu.VMEM((V_TAIL * EMBED // 128, 128), jnp.float32),
        pltpu.SemaphoreType.DMA,
        pltpu.SemaphoreType.DMA,
    ],
)
def _transpose_kernel(
    tab_hbm, scr_hbm, in_v, out_v, ex_in, ex_out, tail_in, tail_out, isem, osem
):
    wid = lax.axis_index("s") * NC + lax.axis_index("c")
    wbase = wid * T_PER_W
    iota = _iota16()

    def in_src(it):
        return tab_hbm.at[:, pl.ds((wbase + it * G) * 128, G * 128)]

    def out_dst(it):
        return scr_hbm.at[pl.ds((wbase + it * G) * 32, G * 32), :]

    for w in range(3):
        pltpu.async_copy(in_src(w), in_v.at[w], isem)

    @pl.loop(0, NIT, step=4)
    def _steps(o):
        for p in range(4):
            it = o + p
            pltpu.make_async_copy(in_src(it), in_v.at[p], isem).wait()

            @pl.when(it + 3 < NIT)
            def _prefetch():
                pltpu.async_copy(in_src(it + 3), in_v.at[(p + 3) % 4], isem)

            @pl.when(it >= 4)
            def _drain():
                pltpu.make_async_copy(out_v.at[p], out_dst(it - 4), osem).wait()

            src = in_v.at[p]
            dst = out_v.at[p]

            def _diag(d, _src=src, _dst=dst):
                perm = (iota + d) & 15
                drow_b = perm >> 2
                dcol_b = 32 * (perm & 3) + iota
                for hh in range(2):
                    rowv = iota + 16 * hh
                    dcol = dcol_b + 16 * hh
                    for vb in range(8):
                        v = plsc.load_gather(_src, [rowv, perm + 16 * vb])
                        plsc.store_scatter(_dst, [drow_b + 4 * vb, dcol], v)

            plsc.parallel_loop(0, 16, unroll=2)(_diag)
            pltpu.async_copy(out_v.at[p], out_dst(it), osem)

    for p in range(4):
        pltpu.make_async_copy(out_v.at[p], out_dst(NIT - 4 + p), osem).wait()

    @pl.when(wid < T_EXTRA)
    def _extra():
        t = NW * T_PER_W + wid
        pltpu.sync_copy(tab_hbm.at[:, pl.ds(t * 128, 128)], ex_in)
        _transpose_block(ex_in, ex_out, 32, 0, iota)
        pltpu.sync_copy(ex_out, scr_hbm.at[pl.ds(t * 32, 32), :])

    @pl.when(wid == T_EXTRA)
    def _tail():
        pltpu.sync_copy(tab_hbm.at[:, pl.ds(NT_FULL * 128, V_TAIL)], tail_in)
        _transpose_block(tail_in, tail_out, V_TAIL * EMBED // 128, 0, iota)
        pltpu.sync_copy(
            tail_out, scr_hbm.at[pl.ds(NT_FULL * 32, V_TAIL * EMBED // 128), :]
        )


@functools.partial(
    pl.kernel,
    mesh=_mesh,
    compiler_params=pltpu.CompilerParams(
        use_tc_tiling_on_sc=False, needs_layout_passes=False
    ),
    out_type=jax.ShapeDtypeStruct((HIST, NFT, NBT, 8, 128), jnp.float32),
    scratch_types=[
        pltpu.VMEM((HIST, BT_PER_W, 128), jnp.int32),
        pltpu.VMEM((4, 128, EMBED), jnp.float32),
        pltpu.VMEM((4, EMBED, 128), jnp.float32),
        pltpu.SemaphoreType.DMA,
        pltpu.SemaphoreType.DMA,
    ],
)
def _gather_kernel(idx_hbm, tab_hbm, out_hbm, idx_v, rows_v, trans_v, gsem, osem):
    wid = lax.axis_index("s") * NC + lax.axis_index("c")
    iota = _iota16()

    pltpu.sync_copy(idx_hbm.at[:, pl.ds(wid * BT_PER_W, BT_PER_W), :], idx_v)

    def gsrc(q):
        return tab_hbm.at[idx_v.at[q // BT_PER_W, q % BT_PER_W]]

    for w in range(3):
        pltpu.async_copy(gsrc(w), rows_v.at[w], gsem)

    @pl.loop(0, NCELL, step=4)
    def _cells(o):
        for p in range(4):
            q = o + p
            h = q // BT_PER_W
            bt = wid * BT_PER_W + q % BT_PER_W
            pltpu.make_async_copy(gsrc(q), rows_v.at[p], gsem).wait()

            @pl.when(q + 3 < NCELL)
            def _prefetch():
                pltpu.async_copy(gsrc(q + 3), rows_v.at[(p + 3) % 4], gsem)

            @pl.when(q >= 4)
            def _drain():
                hd = (q - 4) // BT_PER_W
                btd = wid * BT_PER_W + (q - 4) % BT_PER_W
                for ft in range(NFT):
                    pltpu.make_async_copy(
                        trans_v.at[p].at[pl.ds(8 * ft, 8)],
                        out_hbm.at[hd, ft, btd],
                        osem,
                    ).wait()

            rsrc = rows_v.at[p]
            tdst = trans_v.at[p]

            def _diag(d, _src=rsrc, _dst=tdst):
                perm = (iota + d) & 15
                for hh in range(2):
                    fcols = perm + 16 * hh
                    for bb in range(8):
                        brows = iota + 16 * bb
                        v = plsc.load_gather(_src, [brows, fcols])
                        plsc.store_scatter(_dst, [fcols, brows], v)

            plsc.parallel_loop(0, 16, unroll=2)(_diag)
            for ft in range(NFT):
                pltpu.async_copy(
                    trans_v.at[p].at[pl.ds(8 * ft, 8)], out_hbm.at[h, ft, bt], osem
                )

    for p in range(4):
        q = NCELL - 4 + p
        for ft in range(NFT):
            pltpu.make_async_copy(
                trans_v.at[p].at[pl.ds(8 * ft, 8)],
                out_hbm.at[q // BT_PER_W, ft, wid * BT_PER_W + q % BT_PER_W],
                osem,
            ).wait()


def kernel(indices, table):
    table_t = jnp.swapaxes(table, 0, 1)
    table_lin = _transpose_kernel(table_t).reshape(VOCAB, EMBED)
    idx_lin = (
        jnp.swapaxes(indices, 0, 1).astype(jnp.int32).reshape(HIST, NBT, 128)
    )
    out5 = _gather_kernel(idx_lin, table_lin)
    return out5.transpose(2, 4, 0, 1, 3).reshape(BATCH, HIST, EMBED)


# speedup vs baseline: 3.0098x; 1.1474x over previous
import functools

import jax
import jax.numpy as jnp
from jax import lax
from jax.experimental import pallas as pl
from jax.experimental.pallas import tpu as pltpu
from jax.experimental.pallas import tpu_sc as plsc

BATCH = 16384
HIST = 50
EMBED = 32
VOCAB = 1000000

NC = 2
NS = 16
NW = NC * NS

NBT = BATCH // 128
NFT = EMBED // 8
BT_PER_W = NBT // NW
NCELL = HIST * BT_PER_W

NT_FULL = VOCAB // 128
T_PER_W = NT_FULL // NW
T_EXTRA = NT_FULL - T_PER_W * NW
V_TAIL = VOCAB - NT_FULL * 128

_mesh = plsc.VectorSubcoreMesh(
    core_axis_name="c", subcore_axis_name="s", num_cores=NC, num_subcores=NS
)


def _iota16():
    return lax.iota(jnp.int32, 16)


G = 2
NIT = T_PER_W // G


@functools.partial(
    pl.kernel,
    mesh=_mesh,
    compiler_params=pltpu.CompilerParams(
        use_tc_tiling_on_sc=True, needs_layout_passes=False
    ),
    out_type=jax.ShapeDtypeStruct((VOCAB * EMBED,), jnp.float32),
    scratch_types=[
        pltpu.VMEM((2, EMBED, G * 128), jnp.float32),
        pltpu.VMEM((G * 4096,), jnp.float32),
        pltpu.VMEM((G * 4096,), jnp.float32),
        pltpu.VMEM((EMBED, 128), jnp.float32),
        pltpu.VMEM((4096,), jnp.float32),
        pltpu.VMEM((EMBED, V_TAIL), jnp.float32),
        pltpu.VMEM((V_TAIL * EMBED,), jnp.float32),
        pltpu.SemaphoreType.DMA,
        pltpu.SemaphoreType.DMA,
    ],
)
def _transpose_kernel(
    tab_hbm, scr_hbm, in_v, out_v0, out_v1, ex_in, ex_out, tail_in, tail_out,
    isem, osem
):
    outs = (out_v0, out_v1)
    wid = lax.axis_index("s") * NC + lax.axis_index("c")
    wbase = wid * T_PER_W
    iota = _iota16()

    def in_src(it):
        return tab_hbm.at[:, pl.ds((wbase + it * G) * 128, G * 128)]

    def out_dst(it):
        return scr_hbm.at[pl.ds((wbase + it * G) * 4096, G * 4096)]

    def diag_transpose(src, dst, n_vb):
        def _diag(d):
            perm = (iota + d) & 15
            fvec = (perm << 5) + iota
            for hh in range(2):
                rowv = iota + 16 * hh
                for vb in range(n_vb):
                    v = plsc.load_gather(src, [rowv, perm + 16 * vb])
                    plsc.store_scatter(dst, [fvec + (512 * vb + 16 * hh)], v)

        plsc.parallel_loop(0, 16, unroll=2)(_diag)

    pltpu.async_copy(in_src(0), in_v.at[0], isem)

    @pl.loop(0, NIT, step=2)
    def _steps(o):
        for p in range(2):
            it = o + p
            pltpu.make_async_copy(in_src(it), in_v.at[p], isem).wait()

            @pl.when(it + 1 < NIT)
            def _prefetch():
                pltpu.async_copy(in_src(it + 1), in_v.at[1 - p], isem)

            @pl.when(it >= 2)
            def _drain():
                pltpu.make_async_copy(outs[p], out_dst(it - 2), osem).wait()

            diag_transpose(in_v.at[p], outs[p], G * 8)
            pltpu.async_copy(outs[p], out_dst(it), osem)

    pltpu.make_async_copy(outs[0], out_dst(NIT - 2), osem).wait()
    pltpu.make_async_copy(outs[1], out_dst(NIT - 1), osem).wait()

    @pl.when(wid < T_EXTRA)
    def _extra():
        t = NW * T_PER_W + wid
        pltpu.sync_copy(tab_hbm.at[:, pl.ds(t * 128, 128)], ex_in)
        diag_transpose(ex_in, ex_out, 8)
        pltpu.sync_copy(ex_out, scr_hbm.at[pl.ds(t * 4096, 4096)])

    @pl.when(wid == T_EXTRA)
    def _tail():
        pltpu.sync_copy(tab_hbm.at[:, pl.ds(NT_FULL * 128, V_TAIL)], tail_in)
        diag_transpose(tail_in, tail_out, V_TAIL // 16)
        pltpu.sync_copy(
            tail_out, scr_hbm.at[pl.ds(NT_FULL * 4096, V_TAIL * EMBED)]
        )


@functools.partial(
    pl.kernel,
    mesh=_mesh,
    compiler_params=pltpu.CompilerParams(
        use_tc_tiling_on_sc=False, needs_layout_passes=False
    ),
    out_type=jax.ShapeDtypeStruct((HIST, NFT, NBT, 8, 128), jnp.float32),
    scratch_types=[
        pltpu.VMEM((HIST, BT_PER_W, 128), jnp.int32),
        pltpu.VMEM((4, 128, EMBED), jnp.float32),
        pltpu.VMEM((4, EMBED, 128), jnp.float32),
        pltpu.SemaphoreType.DMA,
        pltpu.SemaphoreType.DMA,
    ],
)
def _gather_kernel(idx_hbm, tab_hbm, out_hbm, idx_v, rows_v, trans_v, gsem, osem):
    wid = lax.axis_index("s") * NC + lax.axis_index("c")
    iota = _iota16()

    pltpu.sync_copy(idx_hbm.at[:, pl.ds(wid * BT_PER_W, BT_PER_W), :], idx_v)

    def gsrc(q):
        return tab_hbm.at[idx_v.at[q // BT_PER_W, q % BT_PER_W]]

    for w in range(3):
        pltpu.async_copy(gsrc(w), rows_v.at[w], gsem)

    @pl.loop(0, NCELL, step=4)
    def _cells(o):
        for p in range(4):
            q = o + p
            h = q // BT_PER_W
            bt = wid * BT_PER_W + q % BT_PER_W
            pltpu.make_async_copy(gsrc(q), rows_v.at[p], gsem).wait()

            @pl.when(q + 3 < NCELL)
            def _prefetch():
                pltpu.async_copy(gsrc(q + 3), rows_v.at[(p + 3) % 4], gsem)

            @pl.when(q >= 4)
            def _drain():
                hd = (q - 4) // BT_PER_W
                btd = wid * BT_PER_W + (q - 4) % BT_PER_W
                for ft in range(NFT):
                    pltpu.make_async_copy(
                        trans_v.at[p].at[pl.ds(8 * ft, 8)],
                        out_hbm.at[hd, ft, btd],
                        osem,
                    ).wait()

            rsrc = rows_v.at[p]
            tdst = trans_v.at[p]

            def _diag(d, _src=rsrc, _dst=tdst):
                perm = (iota + d) & 15
                for hh in range(2):
                    fcols = perm + 16 * hh
                    for bb in range(8):
                        brows = iota + 16 * bb
                        v = plsc.load_gather(_src, [brows, fcols])
                        plsc.store_scatter(_dst, [fcols, brows], v)

            plsc.parallel_loop(0, 16, unroll=2)(_diag)
            for ft in range(NFT):
                pltpu.async_copy(
                    trans_v.at[p].at[pl.ds(8 * ft, 8)], out_hbm.at[h, ft, bt], osem
                )

    for p in range(4):
        q = NCELL - 4 + p
        for ft in range(NFT):
            pltpu.make_async_copy(
                trans_v.at[p].at[pl.ds(8 * ft, 8)],
                out_hbm.at[q // BT_PER_W, ft, wid * BT_PER_W + q % BT_PER_W],
                osem,
            ).wait()


def kernel(indices, table):
    table_t = jnp.swapaxes(table, 0, 1)
    table_lin = _transpose_kernel(table_t).reshape(VOCAB, EMBED)
    idx_lin = (
        jnp.swapaxes(indices, 0, 1).astype(jnp.int32).reshape(HIST, NBT, 128)
    )
    out5 = _gather_kernel(idx_lin, table_lin)
    return out5.transpose(2, 4, 0, 1, 3).reshape(BATCH, HIST, EMBED)


# speedup vs baseline: 3.0268x; 1.0056x over previous
import functools

import jax
import jax.numpy as jnp
from jax import lax
from jax.experimental import pallas as pl
from jax.experimental.pallas import tpu as pltpu
from jax.experimental.pallas import tpu_sc as plsc

BATCH = 16384
HIST = 50
EMBED = 32
VOCAB = 1000000

NC = 2
NS = 16
NW = NC * NS

NBT = BATCH // 128
NFT = EMBED // 8
BT_PER_W = NBT // NW
NCELL = HIST * BT_PER_W

NT_FULL = VOCAB // 128
T_PER_W = NT_FULL // NW
T_EXTRA = NT_FULL - T_PER_W * NW
V_TAIL = VOCAB - NT_FULL * 128

_mesh = plsc.VectorSubcoreMesh(
    core_axis_name="c", subcore_axis_name="s", num_cores=NC, num_subcores=NS
)


def _iota16():
    return lax.iota(jnp.int32, 16)


G = 2
NIT = T_PER_W // G


@functools.partial(
    pl.kernel,
    mesh=_mesh,
    compiler_params=pltpu.CompilerParams(
        use_tc_tiling_on_sc=True, needs_layout_passes=False
    ),
    out_type=jax.ShapeDtypeStruct((VOCAB * EMBED,), jnp.float32),
    scratch_types=[
        pltpu.VMEM((2, EMBED, G * 128), jnp.float32),
        pltpu.VMEM((G * 4096,), jnp.float32),
        pltpu.VMEM((G * 4096,), jnp.float32),
        pltpu.VMEM((EMBED, 128), jnp.float32),
        pltpu.VMEM((4096,), jnp.float32),
        pltpu.VMEM((EMBED, V_TAIL), jnp.float32),
        pltpu.VMEM((V_TAIL * EMBED,), jnp.float32),
        pltpu.SemaphoreType.DMA,
        pltpu.SemaphoreType.DMA,
    ],
)
def _transpose_kernel(
    tab_hbm, scr_hbm, in_v, out_v0, out_v1, ex_in, ex_out, tail_in, tail_out,
    isem, osem
):
    outs = (out_v0, out_v1)
    wid = lax.axis_index("s") * NC + lax.axis_index("c")
    wbase = wid * T_PER_W
    iota = _iota16()

    def in_src(it):
        return tab_hbm.at[:, pl.ds((wbase + it * G) * 128, G * 128)]

    def out_dst(it):
        return scr_hbm.at[pl.ds((wbase + it * G) * 4096, G * 4096)]

    def diag_transpose(src, dst, n_vb):
        def _diag(d):
            perm = (iota + d) & 15
            fvec = (perm << 5) + iota
            for hh in range(2):
                rowv = iota + 16 * hh
                for vb in range(n_vb):
                    v = plsc.load_gather(src, [rowv, perm + 16 * vb])
                    plsc.store_scatter(dst, [fvec + (512 * vb + 16 * hh)], v)

        plsc.parallel_loop(0, 16, unroll=4)(_diag)

    pltpu.async_copy(in_src(0), in_v.at[0], isem)

    @pl.loop(0, NIT, step=2)
    def _steps(o):
        for p in range(2):
            it = o + p
            pltpu.make_async_copy(in_src(it), in_v.at[p], isem).wait()

            @pl.when(it + 1 < NIT)
            def _prefetch():
                pltpu.async_copy(in_src(it + 1), in_v.at[1 - p], isem)

            @pl.when(it >= 2)
            def _drain():
                pltpu.make_async_copy(outs[p], out_dst(it - 2), osem).wait()

            diag_transpose(in_v.at[p], outs[p], G * 8)
            pltpu.async_copy(outs[p], out_dst(it), osem)

    pltpu.make_async_copy(outs[0], out_dst(NIT - 2), osem).wait()
    pltpu.make_async_copy(outs[1], out_dst(NIT - 1), osem).wait()

    @pl.when(wid < T_EXTRA)
    def _extra():
        t = NW * T_PER_W + wid
        pltpu.sync_copy(tab_hbm.at[:, pl.ds(t * 128, 128)], ex_in)
        diag_transpose(ex_in, ex_out, 8)
        pltpu.sync_copy(ex_out, scr_hbm.at[pl.ds(t * 4096, 4096)])

    @pl.when(wid == T_EXTRA)
    def _tail():
        pltpu.sync_copy(tab_hbm.at[:, pl.ds(NT_FULL * 128, V_TAIL)], tail_in)
        diag_transpose(tail_in, tail_out, V_TAIL // 16)
        pltpu.sync_copy(
            tail_out, scr_hbm.at[pl.ds(NT_FULL * 4096, V_TAIL * EMBED)]
        )


@functools.partial(
    pl.kernel,
    mesh=_mesh,
    compiler_params=pltpu.CompilerParams(
        use_tc_tiling_on_sc=False, needs_layout_passes=False
    ),
    out_type=jax.ShapeDtypeStruct((HIST, NFT, NBT, 8, 128), jnp.float32),
    scratch_types=[
        pltpu.VMEM((HIST, BT_PER_W, 128), jnp.int32),
        pltpu.VMEM((4, 128, EMBED), jnp.float32),
        pltpu.VMEM((4, EMBED, 128), jnp.float32),
        pltpu.SemaphoreType.DMA,
        pltpu.SemaphoreType.DMA,
    ],
)
def _gather_kernel(idx_hbm, tab_hbm, out_hbm, idx_v, rows_v, trans_v, gsem, osem):
    wid = lax.axis_index("s") * NC + lax.axis_index("c")
    iota = _iota16()

    pltpu.sync_copy(idx_hbm.at[:, pl.ds(wid * BT_PER_W, BT_PER_W), :], idx_v)

    def gsrc(q):
        return tab_hbm.at[idx_v.at[q // BT_PER_W, q % BT_PER_W]]

    for w in range(3):
        pltpu.async_copy(gsrc(w), rows_v.at[w], gsem)

    @pl.loop(0, NCELL, step=4)
    def _cells(o):
        for p in range(4):
            q = o + p
            h = q // BT_PER_W
            bt = wid * BT_PER_W + q % BT_PER_W
            pltpu.make_async_copy(gsrc(q), rows_v.at[p], gsem).wait()

            @pl.when(q + 3 < NCELL)
            def _prefetch():
                pltpu.async_copy(gsrc(q + 3), rows_v.at[(p + 3) % 4], gsem)

            @pl.when(q >= 4)
            def _drain():
                hd = (q - 4) // BT_PER_W
                btd = wid * BT_PER_W + (q - 4) % BT_PER_W
                for ft in range(NFT):
                    pltpu.make_async_copy(
                        trans_v.at[p].at[pl.ds(8 * ft, 8)],
                        out_hbm.at[hd, ft, btd],
                        osem,
                    ).wait()

            rsrc = rows_v.at[p]
            tdst = trans_v.at[p]

            def _diag(d, _src=rsrc, _dst=tdst):
                perm = (iota + d) & 15
                for hh in range(2):
                    fcols = perm + 16 * hh
                    for bb in range(8):
                        brows = iota + 16 * bb
                        v = plsc.load_gather(_src, [brows, fcols])
                        plsc.store_scatter(_dst, [fcols, brows], v)

            plsc.parallel_loop(0, 16, unroll=2)(_diag)
            for ft in range(NFT):
                pltpu.async_copy(
                    trans_v.at[p].at[pl.ds(8 * ft, 8)], out_hbm.at[h, ft, bt], osem
                )

    for p in range(4):
        q = NCELL - 4 + p
        for ft in range(NFT):
            pltpu.make_async_copy(
                trans_v.at[p].at[pl.ds(8 * ft, 8)],
                out_hbm.at[q // BT_PER_W, ft, wid * BT_PER_W + q % BT_PER_W],
                osem,
            ).wait()


def kernel(indices, table):
    table_t = jnp.swapaxes(table, 0, 1)
    table_lin = _transpose_kernel(table_t).reshape(VOCAB, EMBED)
    idx_lin = (
        jnp.swapaxes(indices, 0, 1).astype(jnp.int32).reshape(HIST, NBT, 128)
    )
    out5 = _gather_kernel(idx_lin, table_lin)
    return out5.transpose(2, 4, 0, 1, 3).reshape(BATCH, HIST, EMBED)


# speedup vs baseline: 3.4844x; 1.1512x over previous
import functools

import jax
import jax.numpy as jnp
from jax import lax
from jax.experimental import pallas as pl
from jax.experimental.pallas import tpu as pltpu
from jax.experimental.pallas import tpu_sc as plsc

BATCH = 16384
HIST = 50
EMBED = 32
VOCAB = 1000000

NC = 2
NS = 16
NW = NC * NS

NBT = BATCH // 128
NFT = EMBED // 8
BT_PER_W = NBT // NW
NCELL = HIST * BT_PER_W

NT_FULL = VOCAB // 128
T_PER_W = NT_FULL // NW
T_EXTRA = NT_FULL - T_PER_W * NW
V_TAIL = VOCAB - NT_FULL * 128

_mesh = plsc.VectorSubcoreMesh(
    core_axis_name="c", subcore_axis_name="s", num_cores=NC, num_subcores=NS
)


def _iota16():
    return lax.iota(jnp.int32, 16)


G = 2
NIT = T_PER_W // G


@functools.partial(
    pl.kernel,
    mesh=_mesh,
    compiler_params=pltpu.CompilerParams(
        use_tc_tiling_on_sc=True, needs_layout_passes=False
    ),
    out_type=jax.ShapeDtypeStruct((VOCAB * EMBED,), jnp.float32),
    scratch_types=[
        pltpu.VMEM((4, EMBED, G * 128), jnp.float32),
        pltpu.VMEM((G * 4096,), jnp.float32),
        pltpu.VMEM((G * 4096,), jnp.float32),
        pltpu.VMEM((G * 4096,), jnp.float32),
        pltpu.VMEM((G * 4096,), jnp.float32),
        pltpu.VMEM((EMBED, 128), jnp.float32),
        pltpu.VMEM((4096,), jnp.float32),
        pltpu.VMEM((EMBED, V_TAIL), jnp.float32),
        pltpu.VMEM((V_TAIL * EMBED,), jnp.float32),
        pltpu.SemaphoreType.DMA,
        pltpu.SemaphoreType.DMA,
    ],
)
def _transpose_kernel(
    tab_hbm, scr_hbm, in_v, out_v0, out_v1, out_v2, out_v3, ex_in, ex_out,
    tail_in, tail_out, isem, osem
):
    outs = (out_v0, out_v1, out_v2, out_v3)
    wid = lax.axis_index("s") * NC + lax.axis_index("c")
    wbase = wid * T_PER_W
    iota = _iota16()

    def in_src(it):
        return tab_hbm.at[:, pl.ds((wbase + it * G) * 128, G * 128)]

    def out_dst(it):
        return scr_hbm.at[pl.ds((wbase + it * G) * 4096, G * 4096)]

    def diag_transpose(src, dst, n_vb):
        def _diag(d):
            perm = (iota + d) & 15
            fvec = (perm << 5) + iota
            for hh in range(2):
                rowv = iota + 16 * hh
                for vb in range(n_vb):
                    v = plsc.load_gather(src, [rowv, perm + 16 * vb])
                    plsc.store_scatter(dst, [fvec + (512 * vb + 16 * hh)], v)

        plsc.parallel_loop(0, 16, unroll=4)(_diag)

    NMAIN = NIT - NIT % 4

    for w in range(3):
        pltpu.async_copy(in_src(w), in_v.at[w], isem)

    @pl.loop(0, NMAIN, step=4)
    def _steps(o):
        for p in range(4):
            it = o + p
            pltpu.make_async_copy(in_src(it), in_v.at[p], isem).wait()

            @pl.when(it + 3 < NMAIN)
            def _prefetch():
                pltpu.async_copy(in_src(it + 3), in_v.at[(p + 3) % 4], isem)

            @pl.when(it >= 4)
            def _drain():
                pltpu.make_async_copy(outs[p], out_dst(it - 4), osem).wait()

            diag_transpose(in_v.at[p], outs[p], G * 8)
            pltpu.async_copy(outs[p], out_dst(it), osem)

    for p in range(4):
        pltpu.make_async_copy(outs[p], out_dst(NMAIN - 4 + p), osem).wait()

    for r in range(NIT % 4):
        pltpu.sync_copy(in_src(NMAIN + r), in_v.at[r])
        diag_transpose(in_v.at[r], outs[r], G * 8)
        pltpu.sync_copy(outs[r], out_dst(NMAIN + r))

    @pl.when(wid < T_EXTRA)
    def _extra():
        t = NW * T_PER_W + wid
        pltpu.sync_copy(tab_hbm.at[:, pl.ds(t * 128, 128)], ex_in)
        diag_transpose(ex_in, ex_out, 8)
        pltpu.sync_copy(ex_out, scr_hbm.at[pl.ds(t * 4096, 4096)])

    @pl.when(wid == T_EXTRA)
    def _tail():
        pltpu.sync_copy(tab_hbm.at[:, pl.ds(NT_FULL * 128, V_TAIL)], tail_in)
        diag_transpose(tail_in, tail_out, V_TAIL // 16)
        pltpu.sync_copy(
            tail_out, scr_hbm.at[pl.ds(NT_FULL * 4096, V_TAIL * EMBED)]
        )


@functools.partial(
    pl.kernel,
    mesh=_mesh,
    compiler_params=pltpu.CompilerParams(
        use_tc_tiling_on_sc=False, needs_layout_passes=False
    ),
    out_type=jax.ShapeDtypeStruct((HIST, NFT, NBT, 8, 128), jnp.float32),
    scratch_types=[
        pltpu.VMEM((HIST, BT_PER_W, 128), jnp.int32),
        pltpu.VMEM((4, 128, EMBED), jnp.float32),
        pltpu.VMEM((4, EMBED, 128), jnp.float32),
        pltpu.SemaphoreType.DMA,
        pltpu.SemaphoreType.DMA,
    ],
)
def _gather_kernel(idx_hbm, tab_hbm, out_hbm, idx_v, rows_v, trans_v, gsem, osem):
    wid = lax.axis_index("s") * NC + lax.axis_index("c")
    iota = _iota16()

    pltpu.sync_copy(idx_hbm.at[:, pl.ds(wid * BT_PER_W, BT_PER_W), :], idx_v)

    def gsrc(q):
        return tab_hbm.at[idx_v.at[q // BT_PER_W, q % BT_PER_W]]

    for w in range(3):
        pltpu.async_copy(gsrc(w), rows_v.at[w], gsem)

    @pl.loop(0, NCELL, step=4)
    def _cells(o):
        for p in range(4):
            q = o + p
            h = q // BT_PER_W
            bt = wid * BT_PER_W + q % BT_PER_W
            pltpu.make_async_copy(gsrc(q), rows_v.at[p], gsem).wait()

            @pl.when(q + 3 < NCELL)
            def _prefetch():
                pltpu.async_copy(gsrc(q + 3), rows_v.at[(p + 3) % 4], gsem)

            @pl.when(q >= 4)
            def _drain():
                hd = (q - 4) // BT_PER_W
                btd = wid * BT_PER_W + (q - 4) % BT_PER_W
                for ft in range(NFT):
                    pltpu.make_async_copy(
                        trans_v.at[p].at[pl.ds(8 * ft, 8)],
                        out_hbm.at[hd, ft, btd],
                        osem,
                    ).wait()

            rsrc = rows_v.at[p]
            tdst = trans_v.at[p]

            def _diag(d, _src=rsrc, _dst=tdst):
                perm = (iota + d) & 15
                for hh in range(2):
                    fcols = perm + 16 * hh
                    for bb in range(8):
                        brows = iota + 16 * bb
                        v = plsc.load_gather(_src, [brows, fcols])
                        plsc.store_scatter(_dst, [fcols, brows], v)

            plsc.parallel_loop(0, 16, unroll=2)(_diag)
            for ft in range(NFT):
                pltpu.async_copy(
                    trans_v.at[p].at[pl.ds(8 * ft, 8)], out_hbm.at[h, ft, bt], osem
                )

    for p in range(4):
        q = NCELL - 4 + p
        for ft in range(NFT):
            pltpu.make_async_copy(
                trans_v.at[p].at[pl.ds(8 * ft, 8)],
                out_hbm.at[q // BT_PER_W, ft, wid * BT_PER_W + q % BT_PER_W],
                osem,
            ).wait()


def kernel(indices, table):
    table_t = jnp.swapaxes(table, 0, 1)
    table_lin = _transpose_kernel(table_t).reshape(VOCAB, EMBED)
    idx_lin = (
        jnp.swapaxes(indices, 0, 1).astype(jnp.int32).reshape(HIST, NBT, 128)
    )
    out5 = _gather_kernel(idx_lin, table_lin)
    return out5.transpose(2, 4, 0, 1, 3).reshape(BATCH, HIST, EMBED)
